# Initial kernel scaffold; baseline (speedup 1.0000x reference)
import jax
import jax.numpy as jnp
from jax.experimental import pallas as pl


def kernel(x, pos, edge_index, edge_attr, W_in, b_in, W_pre, b_pre, W_post, b_post, W_r1, b_r1, W_r2, b_r2):
    raise NotImplementedError("write your pallas kernel here")


# speedup vs baseline: 4.6726x; 4.6726x over previous
import functools

import jax
import jax.numpy as jnp
from jax import lax
from jax.experimental import pallas as pl
from jax.experimental.pallas import tpu as pltpu
from jax.experimental.pallas import tpu_sc as plsc

_NC = 2
_NS = 16
_NW = _NC * _NS
_K = 128


def _ceil_to(a, m):
  return (a + m - 1) // m * m


def _make_layer_sc(n_pad, e_pad, d):
  ept = e_pad // _NW
  nchunk = ept // _K
  zr = n_pad // _NS
  mesh = plsc.VectorSubcoreMesh(core_axis_name="c", subcore_axis_name="s")

  @functools.partial(
      pl.kernel,
      out_type=jax.ShapeDtypeStruct((_NC * n_pad, d), jnp.float32),
      mesh=mesh,
      scratch_types=[
          pltpu.VMEM((_K,), jnp.int32),
          pltpu.VMEM((_K,), jnp.int32),
          pltpu.VMEM((_K, d), jnp.float32),
          pltpu.VMEM_SHARED((n_pad, d), jnp.float32),
          pltpu.SemaphoreType.DMA,
      ],
  )
  def layer_sc(h_hbm, src_hbm, dst_hbm, zero_hbm, out_hbm,
               src_v, dst_v, rows_v, acc_sh, sem):
    cid = lax.axis_index("c")
    sid = lax.axis_index("s")
    wid = sid * _NC + cid
    pltpu.sync_copy(zero_hbm.at[pl.ds(sid * zr, zr)], acc_sh.at[pl.ds(sid * zr, zr)])
    plsc.subcore_barrier()
    base0 = wid * ept

    def body(i, _):
      base = base0 + i * _K
      pltpu.sync_copy(src_hbm.at[pl.ds(base, _K)], src_v)
      pltpu.sync_copy(dst_hbm.at[pl.ds(base, _K)], dst_v)
      pltpu.async_copy(h_hbm.at[src_v], rows_v, sem).wait()
      pltpu.sync_copy(rows_v, acc_sh.at[dst_v], add=True)
      return 0

    lax.fori_loop(0, nchunk, body, 0)
    plsc.subcore_barrier()
    pltpu.sync_copy(acc_sh.at[pl.ds(sid * zr, zr)],
                    out_hbm.at[pl.ds(cid * n_pad + sid * zr, zr)])

  return layer_sc


def _make_pre_sc(n_pad, e_pad, w):
  ept = e_pad // _NW
  nchunk = ept // _K
  zr = n_pad // _NS
  mesh = plsc.VectorSubcoreMesh(core_axis_name="c", subcore_axis_name="s")

  @functools.partial(
      pl.kernel,
      out_type=(jax.ShapeDtypeStruct((_NC * n_pad, w), jnp.float32),
                jax.ShapeDtypeStruct((_NC * n_pad, w), jnp.float32)),
      mesh=mesh,
      compiler_params=pltpu.CompilerParams(use_tc_tiling_on_sc=False),
      scratch_types=[
          pltpu.VMEM((_K,), jnp.int32),
          pltpu.VMEM((_K,), jnp.int32),
          pltpu.VMEM((_K, w), jnp.float32),
          pltpu.VMEM((_K, w), jnp.float32),
          pltpu.VMEM_SHARED((n_pad, w), jnp.float32),
          pltpu.VMEM_SHARED((n_pad, w), jnp.float32),
          pltpu.SemaphoreType.DMA,
      ],
  )
  def pre_sc(posq_hbm, ea_hbm, src_hbm, dst_hbm, zero_hbm, outa_hbm, outb_hbm,
             src_v, dst_v, rowsa_v, rowsb_v, acca_sh, accb_sh, sem):
    cid = lax.axis_index("c")
    sid = lax.axis_index("s")
    wid = sid * _NC + cid
    pltpu.sync_copy(zero_hbm.at[pl.ds(sid * zr, zr)], acca_sh.at[pl.ds(sid * zr, zr)])
    pltpu.sync_copy(zero_hbm.at[pl.ds(sid * zr, zr)], accb_sh.at[pl.ds(sid * zr, zr)])
    plsc.subcore_barrier()
    base0 = wid * ept

    def body(i, _):
      base = base0 + i * _K
      pltpu.sync_copy(src_hbm.at[pl.ds(base, _K)], src_v)
      pltpu.sync_copy(dst_hbm.at[pl.ds(base, _K)], dst_v)
      pltpu.async_copy(posq_hbm.at[src_v], rowsa_v, sem).wait()
      pltpu.sync_copy(rowsa_v, acca_sh.at[dst_v], add=True)
      pltpu.sync_copy(ea_hbm.at[pl.ds(base, _K)], rowsb_v)
      pltpu.sync_copy(rowsb_v, accb_sh.at[dst_v], add=True)
      return 0

    lax.fori_loop(0, nchunk, body, 0)
    plsc.subcore_barrier()
    pltpu.sync_copy(acca_sh.at[pl.ds(sid * zr, zr)],
                    outa_hbm.at[pl.ds(cid * n_pad + sid * zr, zr)])
    pltpu.sync_copy(accb_sh.at[pl.ds(sid * zr, zr)],
                    outb_hbm.at[pl.ds(cid * n_pad + sid * zr, zr)])

  return pre_sc


def _full_spec(shape):
  nd = len(shape)
  return pl.BlockSpec(shape, lambda i, _n=nd: (0,) * _n)


def _posq_tc(pos):
  n = pos.shape[0]
  r = 1000
  grid = (n // r,)

  def body(pos_r, out_r):
    p = pos_r[...]
    q = jnp.sum(p * p, axis=1, keepdims=True)
    one = jnp.ones_like(q)
    zpad = jnp.zeros((p.shape[0], 11), jnp.float32)
    out_r[...] = jnp.concatenate([p, q, one, zpad], axis=1)

  return pl.pallas_call(
      body,
      grid=grid,
      in_specs=[pl.BlockSpec((r, 3), lambda i: (i, 0))],
      out_specs=pl.BlockSpec((r, 16), lambda i: (i, 0)),
      out_shape=jax.ShapeDtypeStruct((n, 16), jnp.float32),
  )(pos)


def _prep_tc(x, w_in, b_in, sa0, sa1, sb0, sb1, posq):
  n, d = x.shape
  r = 1000
  grid = (n // r,)

  def body(x_r, w_r, b_r, sa0_r, sa1_r, sb0_r, sb1_r, pq_r, h_r, ea_r, aux_r):
    h = jnp.dot(x_r[...], w_r[...], preferred_element_type=jnp.float32) + b_r[...]
    h_r[...] = jnp.maximum(h, 0.0)
    sav = sa0_r[...] + sa1_r[...]
    ea_r[...] = sb0_r[...] + sb1_r[...]
    pq = pq_r[...]
    psum = sav[:, 0:3]
    qsum = sav[:, 3:4]
    deg = sav[:, 4:5]
    p = pq[:, 0:3]
    q = pq[:, 3:4]
    sqd = qsum - 2.0 * jnp.sum(p * psum, axis=1, keepdims=True) + deg * q
    aux_r[...] = jnp.concatenate(
        [deg, sqd, jnp.zeros((deg.shape[0], 6), jnp.float32)], axis=1)

  return pl.pallas_call(
      body,
      grid=grid,
      in_specs=[
          pl.BlockSpec((r, d), lambda i: (i, 0)),
          _full_spec((d, d)),
          _full_spec((1, d)),
          pl.BlockSpec((r, 16), lambda i: (i, 0)),
          pl.BlockSpec((r, 16), lambda i: (i, 0)),
          pl.BlockSpec((r, 16), lambda i: (i, 0)),
          pl.BlockSpec((r, 16), lambda i: (i, 0)),
          pl.BlockSpec((r, 16), lambda i: (i, 0)),
      ],
      out_specs=[
          pl.BlockSpec((r, d), lambda i: (i, 0)),
          pl.BlockSpec((r, 16), lambda i: (i, 0)),
          pl.BlockSpec((r, 8), lambda i: (i, 0)),
      ],
      out_shape=[
          jax.ShapeDtypeStruct((n, d), jnp.float32),
          jax.ShapeDtypeStruct((n, 16), jnp.float32),
          jax.ShapeDtypeStruct((n, 8), jnp.float32),
      ],
  )(x, w_in, b_in, sa0, sa1, sb0, sb1, posq)


def _post_tc(h, s0, s1, ea, aux, w_a, w_b, w_c, w_d, b_pre, w_1, w_2, b_post):
  n, d = h.shape
  de = ea.shape[1]
  r = 1000
  grid = (n // r,)

  def body(h_r, s0_r, s1_r, ea_r, aux_r, wa_r, wb_r, wc_r, wd_r, bpre_r,
           w1_r, w2_r, bpost_r, out_r):
    h_v = h_r[...]
    s_v = s0_r[...] + s1_r[...]
    deg = aux_r[:, 0:1]
    sqd = aux_r[:, 1:2]
    m = jnp.dot(s_v, wa_r[...], preferred_element_type=jnp.float32)
    m += jnp.dot(deg * h_v, wb_r[...], preferred_element_type=jnp.float32)
    m += jnp.dot(ea_r[...], wc_r[...], preferred_element_type=jnp.float32)
    m += sqd * wd_r[...]
    m += deg * bpre_r[...]
    out = h_v + jnp.dot(h_v, w1_r[...], preferred_element_type=jnp.float32)
    out += jnp.dot(m, w2_r[...], preferred_element_type=jnp.float32)
    out += bpost_r[...]
    out_r[...] = out

  return pl.pallas_call(
      body,
      grid=grid,
      in_specs=[
          pl.BlockSpec((r, d), lambda i: (i, 0)),
          pl.BlockSpec((r, d), lambda i: (i, 0)),
          pl.BlockSpec((r, d), lambda i: (i, 0)),
          pl.BlockSpec((r, de), lambda i: (i, 0)),
          pl.BlockSpec((r, 8), lambda i: (i, 0)),
          _full_spec((d, d)),
          _full_spec((d, d)),
          _full_spec((de, d)),
          _full_spec((1, d)),
          _full_spec((1, d)),
          _full_spec((d, d)),
          _full_spec((d, d)),
          _full_spec((1, d)),
      ],
      out_specs=pl.BlockSpec((r, d), lambda i: (i, 0)),
      out_shape=jax.ShapeDtypeStruct((n, d), jnp.float32),
  )(h, s0, s1, ea, aux, w_a, w_b, w_c, w_d, b_pre, w_1, w_2, b_post)


def _readout_tc(h, w_r1, b_r1, w_r2, b_r2):
  n, d = h.shape
  t = w_r2.shape[1]

  def body(h_r, w1_r, b1_r, w2_r, b2_r, o_r):
    h_v = h_r[...]
    mean = jnp.mean(h_v, axis=0, keepdims=True)
    mx = jnp.max(h_v, axis=0, keepdims=True)
    mm = jnp.concatenate([mean, mx], axis=1)
    mm8 = jnp.broadcast_to(mm, (8, 2 * d))
    r1 = jnp.dot(mm8, w1_r[...], preferred_element_type=jnp.float32) + b1_r[...]
    r1 = jnp.maximum(r1, 0.0)
    r2 = jnp.dot(r1, w2_r[...], preferred_element_type=jnp.float32) + b2_r[...]
    o_r[...] = r2[0:1, :]

  return pl.pallas_call(
      body,
      in_specs=[
          pl.BlockSpec(h.shape, lambda: (0, 0)),
          pl.BlockSpec(w_r1.shape, lambda: (0, 0)),
          pl.BlockSpec((1, w_r1.shape[1]), lambda: (0, 0)),
          pl.BlockSpec(w_r2.shape, lambda: (0, 0)),
          pl.BlockSpec((1, t), lambda: (0, 0)),
      ],
      out_specs=pl.BlockSpec((1, t), lambda: (0, 0)),
      out_shape=jax.ShapeDtypeStruct((1, t), jnp.float32),
  )(h, w_r1, b_r1.reshape(1, -1), w_r2, b_r2.reshape(1, -1))


def kernel(x, pos, edge_index, edge_attr, W_in, b_in, W_pre, b_pre,
           W_post, b_post, W_r1, b_r1, W_r2, b_r2):
  n, d = x.shape
  e = edge_index.shape[1]
  de = edge_attr.shape[1]
  nl = W_pre.shape[0]

  e_pad = _ceil_to(e, _NW * _K)
  n_pad = _ceil_to(n + 1, _NS * 8)
  pad = e_pad - e
  src = jnp.concatenate([edge_index[0].astype(jnp.int32),
                         jnp.zeros((pad,), jnp.int32)])
  dst = jnp.concatenate([edge_index[1].astype(jnp.int32),
                         jnp.full((pad,), n, jnp.int32)])
  ea_p = jnp.concatenate([edge_attr, jnp.zeros((pad, de), jnp.float32)])
  zero16 = jnp.zeros((n_pad, 16), jnp.float32)
  zero_d = jnp.zeros((n_pad, d), jnp.float32)

  posq = _posq_tc(pos)
  sa, sb = _make_pre_sc(n_pad, e_pad, 16)(posq, ea_p, src, dst, zero16)
  h, ea, aux = _prep_tc(x, W_in, b_in.reshape(1, -1),
                        sa[:n], sa[n_pad:n_pad + n],
                        sb[:n], sb[n_pad:n_pad + n], posq)

  layer_sc = _make_layer_sc(n_pad, e_pad, d)
  for l in range(nl):
    s = layer_sc(h, src, dst, zero_d)
    h = _post_tc(h, s[:n], s[n_pad:n_pad + n], ea, aux,
                 W_pre[l, 0:d], W_pre[l, d:2 * d], W_pre[l, 2 * d:2 * d + de],
                 W_pre[l, 2 * d + de:2 * d + de + 1], b_pre[l].reshape(1, -1),
                 W_post[l, 0:d], W_post[l, d:2 * d], b_post[l].reshape(1, -1))

  return _readout_tc(h, W_r1, b_r1, W_r2, b_r2)


# speedup vs baseline: 6.0022x; 1.2845x over previous
import functools

import jax
import jax.numpy as jnp
from jax import lax
from jax.experimental import pallas as pl
from jax.experimental.pallas import tpu as pltpu
from jax.experimental.pallas import tpu_sc as plsc

_NC = 2
_NS = 16
_NW = _NC * _NS
_K = 120
_NBUF = 3

_SC_PARAMS = pltpu.CompilerParams(use_tc_tiling_on_sc=False)
_MESH = dict(core_axis_name="c", subcore_axis_name="s")


def _ceil_to(a, m):
  return (a + m - 1) // m * m


def _rb(a):
  return a.astype(jnp.bfloat16).astype(jnp.float32)


def _split3(a):
  a0 = _rb(a)
  r1 = a - a0
  a1 = _rb(r1)
  a2 = r1 - a1
  return a0, a1, a2


def _hdot(a, b):
  a0, a1, a2 = _split3(a)
  b0, b1, b2 = _split3(b)

  def _d(u, v):
    return jnp.dot(u, v, preferred_element_type=jnp.float32)

  return (_d(a0, b0) + (_d(a0, b1) + _d(a1, b0))
          + (_d(a0, b2) + _d(a1, b1) + _d(a2, b0)))


def _make_layer_sc(n_pad, e_pad, d):
  ept = e_pad // _NW
  nchunk = ept // _K
  ngroup = nchunk // _NBUF
  zr = n_pad // _NS
  mesh = plsc.VectorSubcoreMesh(**_MESH)

  @functools.partial(
      pl.kernel,
      out_type=jax.ShapeDtypeStruct((_NC, n_pad, d), jnp.float32),
      mesh=mesh,
      compiler_params=_SC_PARAMS,
      scratch_types=[
          pltpu.VMEM((_NBUF, _K), jnp.int32),
          pltpu.VMEM((_NBUF, _K), jnp.int32),
          pltpu.VMEM((_NBUF, _K, d), jnp.float32),
          pltpu.VMEM_SHARED((n_pad, d), jnp.float32),
          pltpu.SemaphoreType.DMA((_NBUF,)),
          pltpu.SemaphoreType.DMA((_NBUF,)),
          pltpu.SemaphoreType.DMA((_NBUF,)),
          pltpu.SemaphoreType.DMA((_NBUF,)),
      ],
  )
  def layer_sc(h_hbm, src_hbm, dst_hbm, zero_hbm, out_hbm,
               srcs_v, dsts_v, bufs, acc_sh, isem, jsem, gsem, ssem):
    cid = lax.axis_index("c")
    sid = lax.axis_index("s")
    wid = sid * _NC + cid
    row0 = wid * nchunk

    def i_desc(i, b):
      return pltpu.make_async_copy(src_hbm.at[row0 + i], srcs_v.at[b], isem.at[b])

    def j_desc(i, b):
      return pltpu.make_async_copy(dst_hbm.at[row0 + i], dsts_v.at[b], jsem.at[b])

    def g_desc(b):
      return pltpu.make_async_copy(h_hbm.at[srcs_v.at[b]], bufs.at[b], gsem.at[b])

    def s_desc(b):
      return pltpu.make_async_copy(bufs.at[b], acc_sh.at[dsts_v.at[b]], ssem.at[b])

    for b in range(_NBUF):
      i_desc(b, b).start()
      j_desc(b, b).start()
    pltpu.sync_copy(zero_hbm.at[pl.ds(sid * zr, zr)], acc_sh.at[pl.ds(sid * zr, zr)])
    plsc.subcore_barrier()
    for b in range(_NBUF):
      i_desc(b, b).wait()
      g_desc(b).start()

    def group(g, _):
      for b in range(_NBUF):
        i = g * _NBUF + b
        g_desc(b).wait()
        j_desc(i, b).wait()
        s_desc(b).start(add=True)
        nxt = i + _NBUF

        @pl.when(nxt < nchunk)
        def _():
          i_desc(nxt, b).start()
          s_desc(b).wait()
          j_desc(nxt, b).start()
          i_desc(nxt, b).wait()
          g_desc(b).start()

      return 0

    lax.fori_loop(0, ngroup, group, 0)
    for b in range(_NBUF):
      s_desc(b).wait()
    plsc.subcore_barrier()
    pltpu.sync_copy(acc_sh.at[pl.ds(sid * zr, zr)],
                    out_hbm.at[cid, pl.ds(sid * zr, zr)])

  return layer_sc


def _make_gather_sc(e_pad, w):
  ept = e_pad // _NW
  nchunk = ept // _K
  ngroup = nchunk // _NBUF
  mesh = plsc.VectorSubcoreMesh(**_MESH)

  @functools.partial(
      pl.kernel,
      out_type=jax.ShapeDtypeStruct((e_pad, w), jnp.float32),
      mesh=mesh,
      compiler_params=_SC_PARAMS,
      scratch_types=[
          pltpu.VMEM((_NBUF, _K), jnp.int32),
          pltpu.VMEM((_NBUF, _K, w), jnp.float32),
          pltpu.SemaphoreType.DMA((_NBUF,)),
          pltpu.SemaphoreType.DMA((_NBUF,)),
          pltpu.SemaphoreType.DMA((_NBUF,)),
      ],
  )
  def gather_sc(tab_hbm, idx_hbm, out_hbm, idxs_v, bufs, isem, gsem, wsem):
    cid = lax.axis_index("c")
    sid = lax.axis_index("s")
    wid = sid * _NC + cid
    row0 = wid * nchunk
    base0 = wid * ept

    def i_desc(i, b):
      return pltpu.make_async_copy(idx_hbm.at[row0 + i], idxs_v.at[b], isem.at[b])

    def g_desc(b):
      return pltpu.make_async_copy(tab_hbm.at[idxs_v.at[b]], bufs.at[b], gsem.at[b])

    def w_desc(i, b):
      return pltpu.make_async_copy(bufs.at[b],
                                   out_hbm.at[pl.ds(base0 + i * _K, _K)],
                                   wsem.at[b])

    for b in range(_NBUF):
      i_desc(b, b).start()
    for b in range(_NBUF):
      i_desc(b, b).wait()
      g_desc(b).start()

    def group(g, _):
      for b in range(_NBUF):
        i = g * _NBUF + b
        g_desc(b).wait()
        w_desc(i, b).start()
        nxt = i + _NBUF

        @pl.when(nxt < nchunk)
        def _():
          i_desc(nxt, b).start()
          w_desc(i, b).wait()
          i_desc(nxt, b).wait()
          g_desc(b).start()

      return 0

    lax.fori_loop(0, ngroup, group, 0)
    for b in range(_NBUF):
      w_desc(nchunk - _NBUF + b, b).wait()

  return gather_sc


def _make_scatter_sc(n_pad, e_pad, w):
  ept = e_pad // _NW
  nchunk = ept // _K
  ngroup = nchunk // _NBUF
  zr = n_pad // _NS
  mesh = plsc.VectorSubcoreMesh(**_MESH)

  @functools.partial(
      pl.kernel,
      out_type=jax.ShapeDtypeStruct((_NC, n_pad, w), jnp.float32),
      mesh=mesh,
      compiler_params=_SC_PARAMS,
      scratch_types=[
          pltpu.VMEM((_NBUF, _K), jnp.int32),
          pltpu.VMEM((_NBUF, _K, w), jnp.float32),
          pltpu.VMEM_SHARED((n_pad, w), jnp.float32),
          pltpu.SemaphoreType.DMA((_NBUF,)),
          pltpu.SemaphoreType.DMA((_NBUF,)),
          pltpu.SemaphoreType.DMA((_NBUF,)),
      ],
  )
  def scatter_sc(pay_hbm, dst_hbm, zero_hbm, out_hbm,
                 dsts_v, bufs, acc_sh, jsem, psem, ssem):
    cid = lax.axis_index("c")
    sid = lax.axis_index("s")
    wid = sid * _NC + cid
    row0 = wid * nchunk
    base0 = wid * ept

    def j_desc(i, b):
      return pltpu.make_async_copy(dst_hbm.at[row0 + i], dsts_v.at[b], jsem.at[b])

    def p_desc(i, b):
      return pltpu.make_async_copy(pay_hbm.at[pl.ds(base0 + i * _K, _K)],
                                   bufs.at[b], psem.at[b])

    def s_desc(b):
      return pltpu.make_async_copy(bufs.at[b], acc_sh.at[dsts_v.at[b]], ssem.at[b])

    for b in range(_NBUF):
      j_desc(b, b).start()
      p_desc(b, b).start()
    pltpu.sync_copy(zero_hbm.at[pl.ds(sid * zr, zr)], acc_sh.at[pl.ds(sid * zr, zr)])
    plsc.subcore_barrier()

    def group(g, _):
      for b in range(_NBUF):
        i = g * _NBUF + b
        p_desc(i, b).wait()
        j_desc(i, b).wait()
        s_desc(b).start(add=True)
        nxt = i + _NBUF

        @pl.when(nxt < nchunk)
        def _():
          s_desc(b).wait()
          j_desc(nxt, b).start()
          p_desc(nxt, b).start()

      return 0

    lax.fori_loop(0, ngroup, group, 0)
    for b in range(_NBUF):
      s_desc(b).wait()
    plsc.subcore_barrier()
    pltpu.sync_copy(acc_sh.at[pl.ds(sid * zr, zr)],
                    out_hbm.at[cid, pl.ds(sid * zr, zr)])

  return scatter_sc


def _full_spec(shape):
  nd = len(shape)
  return pl.BlockSpec(shape, lambda i, _n=nd: (0,) * _n)


def _posq_tc(pos_p):
  n_pad = pos_p.shape[0]
  r = 1264
  grid = (n_pad // r,)

  def body(pos_r, out_r):
    p = pos_r[...]
    zpad = jnp.zeros((p.shape[0], 13), jnp.float32)
    out_r[...] = jnp.concatenate([p, zpad], axis=1)

  return pl.pallas_call(
      body,
      grid=grid,
      in_specs=[pl.BlockSpec((r, 3), lambda i: (i, 0))],
      out_specs=pl.BlockSpec((r, 16), lambda i: (i, 0)),
      out_shape=jax.ShapeDtypeStruct((n_pad, 16), jnp.float32),
  )(pos_p)


def _payload_tc(g1, g2, ea_p):
  e_pad = g1.shape[0]
  de = ea_p.shape[1]
  r = 5040
  grid = (e_pad // r,)

  def body(g1_r, g2_r, ea_r, out_r):
    dlt = g1_r[:, 0:3] - g2_r[:, 0:3]
    sqd = jnp.sum(dlt * dlt, axis=1, keepdims=True)
    one = jnp.ones_like(sqd)
    zpad = jnp.zeros((sqd.shape[0], 32 - 2 - de), jnp.float32)
    out_r[...] = jnp.concatenate([_rb(sqd), one, _rb(ea_r[...]), zpad], axis=1)

  return pl.pallas_call(
      body,
      grid=grid,
      in_specs=[
          pl.BlockSpec((r, 16), lambda i: (i, 0)),
          pl.BlockSpec((r, 16), lambda i: (i, 0)),
          pl.BlockSpec((r, de), lambda i: (i, 0)),
      ],
      out_specs=pl.BlockSpec((r, 32), lambda i: (i, 0)),
      out_shape=jax.ShapeDtypeStruct((e_pad, 32), jnp.float32),
  )(g1, g2, ea_p)


def _prep_tc(x, w_in, b_in, sa):
  n, d = x.shape
  r = 1000
  grid = (n // r,)

  def body(x_r, w_r, b_r, sa0_r, sa1_r, h_r, hb_r, ea_r, aux_r):
    h = _hdot(_rb(x_r[...]), _rb(w_r[...])) + b_r[...]
    h = jnp.maximum(h, 0.0)
    h_r[...] = h
    hb_r[...] = _rb(h)
    sav = sa0_r[0] + sa1_r[0]
    ea_r[...] = sav[:, 2:18]
    sqd = sav[:, 0:1]
    deg = sav[:, 1:2]
    aux_r[...] = jnp.concatenate(
        [deg, sqd, jnp.zeros((deg.shape[0], 6), jnp.float32)], axis=1)

  return pl.pallas_call(
      body,
      grid=grid,
      in_specs=[
          pl.BlockSpec((r, d), lambda i: (i, 0)),
          _full_spec((d, d)),
          _full_spec((1, d)),
          pl.BlockSpec((1, r, 32), lambda i: (0, i, 0)),
          pl.BlockSpec((1, r, 32), lambda i: (1, i, 0)),
      ],
      out_specs=[
          pl.BlockSpec((r, d), lambda i: (i, 0)),
          pl.BlockSpec((r, d), lambda i: (i, 0)),
          pl.BlockSpec((r, 16), lambda i: (i, 0)),
          pl.BlockSpec((r, 8), lambda i: (i, 0)),
      ],
      out_shape=[
          jax.ShapeDtypeStruct((n, d), jnp.float32),
          jax.ShapeDtypeStruct((n, d), jnp.float32),
          jax.ShapeDtypeStruct((n, 16), jnp.float32),
          jax.ShapeDtypeStruct((n, 8), jnp.float32),
      ],
  )(x, w_in, b_in, sa, sa)


def _post_tc(h, hb, s, ea, aux, w_a, w_b, w_c, w_d, b_pre, w_1, w_2, b_post):
  n, d = h.shape
  de = ea.shape[1]
  r = 1000
  grid = (n // r,)

  def body(h_r, hb_r, s0_r, s1_r, ea_r, aux_r, wa_r, wb_r, wc_r, wd_r, bpre_r,
           w1_r, w2_r, bpost_r, out_r, outb_r):
    h_v = h_r[...]
    hb_v = hb_r[...]
    s_v = s0_r[0] + s1_r[0]
    deg = aux_r[:, 0:1]
    sqd = aux_r[:, 1:2]
    m = _hdot(s_v, _rb(wa_r[...]))
    m += deg * _hdot(hb_v, _rb(wb_r[...]))
    m += _hdot(ea_r[...], _rb(wc_r[...]))
    m += sqd * _rb(wd_r[...])
    m += deg * bpre_r[...]
    out = h_v + _hdot(hb_v, _rb(w1_r[...]))
    out += _hdot(_rb(m), _rb(w2_r[...]))
    out += bpost_r[...]
    out_r[...] = out
    outb_r[...] = _rb(out)

  return pl.pallas_call(
      body,
      grid=grid,
      in_specs=[
          pl.BlockSpec((r, d), lambda i: (i, 0)),
          pl.BlockSpec((r, d), lambda i: (i, 0)),
          pl.BlockSpec((1, r, d), lambda i: (0, i, 0)),
          pl.BlockSpec((1, r, d), lambda i: (1, i, 0)),
          pl.BlockSpec((r, de), lambda i: (i, 0)),
          pl.BlockSpec((r, 8), lambda i: (i, 0)),
          _full_spec((d, d)),
          _full_spec((d, d)),
          _full_spec((de, d)),
          _full_spec((1, d)),
          _full_spec((1, d)),
          _full_spec((d, d)),
          _full_spec((d, d)),
          _full_spec((1, d)),
      ],
      out_specs=[
          pl.BlockSpec((r, d), lambda i: (i, 0)),
          pl.BlockSpec((r, d), lambda i: (i, 0)),
      ],
      out_shape=[
          jax.ShapeDtypeStruct((n, d), jnp.float32),
          jax.ShapeDtypeStruct((n, d), jnp.float32),
      ],
  )(h, hb, s, s, ea, aux, w_a, w_b, w_c, w_d, b_pre, w_1, w_2, b_post)


def _readout_tc(h, w_r1, b_r1, w_r2, b_r2):
  n, d = h.shape
  t = w_r2.shape[1]

  def body(h_r, w1_r, b1_r, w2_r, b2_r, o_r):
    h_v = h_r[...]
    mean = jnp.mean(h_v, axis=0, keepdims=True)
    mx = jnp.max(h_v, axis=0, keepdims=True)
    mm = jnp.concatenate([mean, mx], axis=1)
    mm8 = jnp.broadcast_to(mm, (8, 2 * d))
    r1 = _hdot(_rb(mm8), _rb(w1_r[...])) + b1_r[...]
    r1 = jnp.maximum(r1, 0.0)
    r2 = _hdot(_rb(r1), _rb(w2_r[...])) + b2_r[...]
    o_r[...] = r2[0:1, :]

  return pl.pallas_call(
      body,
      in_specs=[
          pl.BlockSpec(h.shape, lambda: (0, 0)),
          pl.BlockSpec(w_r1.shape, lambda: (0, 0)),
          pl.BlockSpec((1, w_r1.shape[1]), lambda: (0, 0)),
          pl.BlockSpec(w_r2.shape, lambda: (0, 0)),
          pl.BlockSpec((1, t), lambda: (0, 0)),
      ],
      out_specs=pl.BlockSpec((1, t), lambda: (0, 0)),
      out_shape=jax.ShapeDtypeStruct((1, t), jnp.float32),
  )(h, w_r1, b_r1.reshape(1, -1), w_r2, b_r2.reshape(1, -1))


def kernel(x, pos, edge_index, edge_attr, W_in, b_in, W_pre, b_pre,
           W_post, b_post, W_r1, b_r1, W_r2, b_r2):
  n, d = x.shape
  e = edge_index.shape[1]
  de = edge_attr.shape[1]
  nl = W_pre.shape[0]

  e_pad = _ceil_to(e, _NW * _K * _NBUF)
  n_pad = _ceil_to(n + 1, _NS * 8)
  ept = e_pad // _NW
  nchunk = ept // _K
  pad = e_pad - e
  src = jnp.concatenate([edge_index[0].astype(jnp.int32),
                         jnp.zeros((pad,), jnp.int32)]).reshape(_NW * nchunk, _K)
  dst = jnp.concatenate([edge_index[1].astype(jnp.int32),
                         jnp.full((pad,), n, jnp.int32)]).reshape(_NW * nchunk, _K)
  ea_p = jnp.concatenate([edge_attr, jnp.zeros((pad, de), jnp.float32)])
  pos_p = jnp.concatenate([pos, jnp.zeros((n_pad - n, 3), jnp.float32)])
  zero32 = jnp.zeros((n_pad, 32), jnp.float32)
  zero_d = jnp.zeros((n_pad, d), jnp.float32)

  posq = _posq_tc(pos_p)
  gather = _make_gather_sc(e_pad, 16)
  g1 = gather(posq, src)
  g2 = gather(posq, dst)
  payload = _payload_tc(g1, g2, ea_p)
  sa = _make_scatter_sc(n_pad, e_pad, 32)(payload, dst, zero32)
  h, hb, ea, aux = _prep_tc(x, W_in, b_in.reshape(1, -1), sa)

  layer_sc = _make_layer_sc(n_pad, e_pad, d)
  for l in range(nl):
    s = layer_sc(hb, src, dst, zero_d)
    h, hb = _post_tc(h, hb, s, ea, aux,
                     W_pre[l, 0:d], W_pre[l, d:2 * d], W_pre[l, 2 * d:2 * d + de],
                     W_pre[l, 2 * d + de:2 * d + de + 1], b_pre[l].reshape(1, -1),
                     W_post[l, 0:d], W_post[l, d:2 * d], b_post[l].reshape(1, -1))

  return _readout_tc(h, W_r1, b_r1, W_r2, b_r2)
